# baseline (speedup 1.0000x reference)
import functools

import jax
import jax.numpy as jnp
from jax.experimental import pallas as pl
from jax.experimental.pallas import tpu as pltpu


def _rvq_body(x_ref, cbf_ref, c2_ref,
              zq_ref, loss_ref, perp_ref, counts_ref,
              hi_ref, lo_ref, lo2_ref, *,
              nq, k, tb, n_tokens):
    pid = pl.program_id(0)
    first = pid == 0
    last = pid == pl.num_programs(0) - 1

    @pl.when(first)
    def _init():
        counts_ref[...] = jnp.zeros_like(counts_ref)
        loss_ref[...] = jnp.zeros_like(loss_ref)
        perp_ref[...] = jnp.zeros_like(perp_ref)
        for s in range(nq):
            cb = cbf_ref[s]
            cb_hi = cb.astype(jnp.bfloat16)
            t_lo = cb - cb_hi.astype(jnp.float32)
            cb_lo = t_lo.astype(jnp.bfloat16)
            hi_ref[s] = cb_hi
            lo_ref[s] = cb_lo
            lo2_ref[s] = (t_lo - cb_lo.astype(jnp.float32)).astype(jnp.bfloat16)

    x = x_ref[...]
    h = tb // 2
    rs = [x[:h], x[h:]]
    accs = [jnp.zeros_like(rs[0]), jnp.zeros_like(rs[1])]
    lsum = jnp.float32(0.0)
    liota = jax.lax.broadcasted_iota(jnp.int32, (h, k), 1)
    ones8 = jnp.ones((8, h), dtype=jnp.bfloat16)
    f2s = [jnp.sum(rs[0] ** 2, axis=1, keepdims=True),
           jnp.sum(rs[1] ** 2, axis=1, keepdims=True)]
    for s in range(nq):
        cnt = None
        for c in range(2):
            r = rs[c]
            sc2 = jax.lax.dot_general(
                (r + r).astype(jnp.bfloat16), hi_ref[s],
                (((1,), (1,)), ((), ())),
                preferred_element_type=jnp.float32)
            d = (f2s[c] - sc2) + c2_ref[s][None, :]
            m = jnp.min(d, axis=1)
            idx = jnp.min(jnp.where(d == m[:, None], liota, k), axis=1)
            onehot = (liota == idx[:, None]).astype(jnp.bfloat16)

            def _pick(mat):
                return jax.lax.dot_general(
                    onehot, mat, (((1,), (0,)), ((), ())),
                    preferred_element_type=jnp.float32)

            zq = (_pick(hi_ref[s]) + _pick(lo_ref[s])) + _pick(lo2_ref[s])
            rs[c] = r - zq
            accs[c] = accs[c] + zq
            f2s[c] = jnp.sum(rs[c] ** 2, axis=1, keepdims=True)
            lsum = lsum + jnp.sum(f2s[c])
            cc = jax.lax.dot_general(
                ones8, onehot, (((1,), (0,)), ((), ())),
                preferred_element_type=jnp.float32)
            cnt = cc if cnt is None else cnt + cc
        counts_ref[s, :] += cnt[0]
    zq_ref[:h, :] = accs[0]
    zq_ref[h:, :] = accs[1]
    loss_ref[...] += jnp.reshape(1.25 * lsum, (1, 1))

    @pl.when(last)
    def _finish():
        p = counts_ref[...] / n_tokens
        ent = -jnp.sum(p * jnp.log(p + 1e-10), axis=1)
        perp_ref[...] = jnp.reshape(jnp.mean(jnp.exp(ent)), (1, 1))


def _rvq_call(z, codebooks, tb, interpret=False):
    B, D, T = z.shape
    NQ, K, _ = codebooks.shape
    N = B * T
    x = jnp.transpose(z, (0, 2, 1)).reshape(N, D)
    c2 = jnp.sum(codebooks ** 2, axis=2)
    grid = (N // tb,)
    body = functools.partial(_rvq_body, nq=NQ, k=K, tb=tb, n_tokens=float(N))
    full = lambda i: (0, 0, 0)
    zq, loss, perp, _ = pl.pallas_call(
        body,
        grid=grid,
        in_specs=[
            pl.BlockSpec((tb, D), lambda i: (i, 0)),
            pl.BlockSpec((NQ, K, D), full),
            pl.BlockSpec((NQ, K), lambda i: (0, 0)),
        ],
        out_specs=[
            pl.BlockSpec((tb, D), lambda i: (i, 0)),
            pl.BlockSpec((1, 1), lambda i: (0, 0)),
            pl.BlockSpec((1, 1), lambda i: (0, 0)),
            pl.BlockSpec((NQ, K), lambda i: (0, 0)),
        ],
        out_shape=[
            jax.ShapeDtypeStruct((N, D), jnp.float32),
            jax.ShapeDtypeStruct((1, 1), jnp.float32),
            jax.ShapeDtypeStruct((1, 1), jnp.float32),
            jax.ShapeDtypeStruct((NQ, K), jnp.float32),
        ],
        scratch_shapes=[
            pltpu.VMEM((NQ, K, D), jnp.bfloat16),
            pltpu.VMEM((NQ, K, D), jnp.bfloat16),
            pltpu.VMEM((NQ, K, D), jnp.bfloat16),
        ],
        interpret=interpret,
    )(x, codebooks, c2)
    zq_out = jnp.transpose(zq.reshape(B, T, D), (0, 2, 1))
    total_loss = (loss[0, 0] / (N * D)).reshape(())
    return zq_out, total_loss, perp[0, 0].reshape(())


def kernel(z, codebooks):
    return _rvq_call(z, codebooks, tb=512)


# speedup vs baseline: 1.0145x; 1.0145x over previous
import functools

import jax
import jax.numpy as jnp
from jax.experimental import pallas as pl
from jax.experimental.pallas import tpu as pltpu


def _rvq_body(x_ref, cbf_ref, c2_ref,
              zq_ref, loss_ref, perp_ref, counts_ref,
              hi_ref, lo_ref, lo2_ref, *,
              nq, k, tb, nc, n_tokens):
    pid = pl.program_id(0)
    first = pid == 0
    last = pid == pl.num_programs(0) - 1

    @pl.when(first)
    def _init():
        counts_ref[...] = jnp.zeros_like(counts_ref)
        loss_ref[...] = jnp.zeros_like(loss_ref)
        perp_ref[...] = jnp.zeros_like(perp_ref)
        for s in range(nq):
            cb = cbf_ref[s]
            cb_hi = cb.astype(jnp.bfloat16)
            t_lo = cb - cb_hi.astype(jnp.float32)
            cb_lo = t_lo.astype(jnp.bfloat16)
            hi_ref[s] = cb_hi
            lo_ref[s] = cb_lo
            lo2_ref[s] = (t_lo - cb_lo.astype(jnp.float32)).astype(jnp.bfloat16)

    x = x_ref[...]
    h = tb // nc
    rs = [x[c * h:(c + 1) * h] for c in range(nc)]
    accs = [jnp.zeros_like(r) for r in rs]
    lsum = jnp.float32(0.0)
    liota = jax.lax.broadcasted_iota(jnp.int32, (h, k), 1)
    ones8 = jnp.ones((8, h), dtype=jnp.bfloat16)
    f2s = [jnp.sum(r ** 2, axis=1, keepdims=True) for r in rs]
    for s in range(nq):
        cnt = None
        for c in range(nc):
            r = rs[c]
            sc2 = jax.lax.dot_general(
                (r + r).astype(jnp.bfloat16), hi_ref[s],
                (((1,), (1,)), ((), ())),
                preferred_element_type=jnp.float32)
            d = (f2s[c] - sc2) + c2_ref[s][None, :]
            m = jnp.min(d, axis=1)
            idx = jnp.min(jnp.where(d == m[:, None], liota, k), axis=1)
            onehot = (liota == idx[:, None]).astype(jnp.bfloat16)

            def _pick(mat):
                return jax.lax.dot_general(
                    onehot, mat, (((1,), (0,)), ((), ())),
                    preferred_element_type=jnp.float32)

            zq = (_pick(hi_ref[s]) + _pick(lo_ref[s])) + _pick(lo2_ref[s])
            rs[c] = r - zq
            accs[c] = accs[c] + zq
            f2s[c] = jnp.sum(rs[c] ** 2, axis=1, keepdims=True)
            lsum = lsum + jnp.sum(f2s[c])
            cc = jax.lax.dot_general(
                ones8, onehot, (((1,), (0,)), ((), ())),
                preferred_element_type=jnp.float32)
            cnt = cc if cnt is None else cnt + cc
        counts_ref[s, :] += cnt[0]
    for c in range(nc):
        zq_ref[c * h:(c + 1) * h, :] = accs[c]
    loss_ref[...] += jnp.reshape(1.25 * lsum, (1, 1))

    @pl.when(last)
    def _finish():
        p = counts_ref[...] / n_tokens
        ent = -jnp.sum(p * jnp.log(p + 1e-10), axis=1)
        perp_ref[...] = jnp.reshape(jnp.mean(jnp.exp(ent)), (1, 1))


def _rvq_call(z, codebooks, tb, nc, interpret=False):
    B, D, T = z.shape
    NQ, K, _ = codebooks.shape
    N = B * T
    x = jnp.transpose(z, (0, 2, 1)).reshape(N, D)
    c2 = jnp.sum(codebooks ** 2, axis=2)
    grid = (N // tb,)
    body = functools.partial(_rvq_body, nq=NQ, k=K, tb=tb, nc=nc,
                             n_tokens=float(N))
    full = lambda i: (0, 0, 0)
    zq, loss, perp, _ = pl.pallas_call(
        body,
        grid=grid,
        in_specs=[
            pl.BlockSpec((tb, D), lambda i: (i, 0)),
            pl.BlockSpec((NQ, K, D), full),
            pl.BlockSpec((NQ, K), lambda i: (0, 0)),
        ],
        out_specs=[
            pl.BlockSpec((tb, D), lambda i: (i, 0)),
            pl.BlockSpec((1, 1), lambda i: (0, 0)),
            pl.BlockSpec((1, 1), lambda i: (0, 0)),
            pl.BlockSpec((NQ, K), lambda i: (0, 0)),
        ],
        out_shape=[
            jax.ShapeDtypeStruct((N, D), jnp.float32),
            jax.ShapeDtypeStruct((1, 1), jnp.float32),
            jax.ShapeDtypeStruct((1, 1), jnp.float32),
            jax.ShapeDtypeStruct((NQ, K), jnp.float32),
        ],
        scratch_shapes=[
            pltpu.VMEM((NQ, K, D), jnp.bfloat16),
            pltpu.VMEM((NQ, K, D), jnp.bfloat16),
            pltpu.VMEM((NQ, K, D), jnp.bfloat16),
        ],
        interpret=interpret,
    )(x, codebooks, c2)
    zq_out = jnp.transpose(zq.reshape(B, T, D), (0, 2, 1))
    total_loss = (loss[0, 0] / (N * D)).reshape(())
    return zq_out, total_loss, perp[0, 0].reshape(())


def kernel(z, codebooks):
    return _rvq_call(z, codebooks, tb=1024, nc=4)
